# Initial kernel scaffold; baseline (speedup 1.0000x reference)
import jax
import jax.numpy as jnp
from jax.experimental import pallas as pl


def kernel(input_ids, token_type_ids, word_emb, type_emb, pos_emb, gamma, beta):
    raise NotImplementedError("write your pallas kernel here")


# speedup vs baseline: 1.7386x; 1.7386x over previous
import functools

import jax
import jax.numpy as jnp
from jax import lax
from jax.experimental import pallas as pl
from jax.experimental.pallas import tpu as pltpu
from jax.experimental.pallas import tpu_sc as plsc

_EPS = 1e-12
_CH = 128


@functools.lru_cache(maxsize=None)
def _sc_gather_fn(V, D, N, NC, NS):
    NW = NC * NS
    b_per_w = N // NW
    nch = b_per_w // _CH
    mesh = plsc.VectorSubcoreMesh(
        core_axis_name="c", subcore_axis_name="s", num_cores=NC, num_subcores=NS
    )

    @functools.partial(
        pl.kernel,
        out_type=jax.ShapeDtypeStruct((N, D), jnp.float32),
        mesh=mesh,
        scratch_types=[
            pltpu.VMEM((nch, _CH), jnp.int32),
            pltpu.VMEM((b_per_w, D), jnp.float32),
            pltpu.SemaphoreType.DMA,
        ],
    )
    def gather_kernel(ids_hbm, table_hbm, out_hbm, idx_v, rows_v, sem):
        wid = lax.axis_index("s") * NC + lax.axis_index("c")
        pltpu.sync_copy(ids_hbm.at[wid], idx_v)
        copies = [
            pltpu.async_copy(
                table_hbm.at[idx_v.at[j]], rows_v.at[pl.ds(j * _CH, _CH)], sem
            )
            for j in range(nch)
        ]
        for cp in copies:
            cp.wait()
        pltpu.sync_copy(rows_v, out_hbm.at[pl.ds(wid * b_per_w, b_per_w)])

    return gather_kernel


@functools.lru_cache(maxsize=None)
def _tc_ln_fn(N, S, D, BLK):
    nblk = N // BLK
    spb = S // BLK

    def body(tt_ref, x_ref, pos_ref, type_ref, g_ref, b_ref, o_ref):
        x = x_ref[...]
        tt = tt_ref[0, 0, :].astype(jnp.float32)[:, None]
        t0 = type_ref[0, :][None, :]
        t1 = type_ref[1, :][None, :]
        x = x + pos_ref[...] + t0 + (t1 - t0) * tt
        mean = jnp.mean(x, axis=-1, keepdims=True)
        xc = x - mean
        var = jnp.mean(xc * xc, axis=-1, keepdims=True)
        y = xc * lax.rsqrt(var + _EPS)
        o_ref[...] = y * g_ref[...] + b_ref[...]

    return pl.pallas_call(
        body,
        grid=(nblk,),
        in_specs=[
            pl.BlockSpec((1, 1, BLK), lambda i: (i, 0, 0)),
            pl.BlockSpec((BLK, D), lambda i: (i, 0)),
            pl.BlockSpec((BLK, D), lambda i: (i % spb, 0)),
            pl.BlockSpec((2, D), lambda i: (0, 0)),
            pl.BlockSpec((1, D), lambda i: (0, 0)),
            pl.BlockSpec((1, D), lambda i: (0, 0)),
        ],
        out_specs=pl.BlockSpec((BLK, D), lambda i: (i, 0)),
        out_shape=jax.ShapeDtypeStruct((N, D), jnp.float32),
    )


def kernel(input_ids, token_type_ids, word_emb, type_emb, pos_emb, gamma, beta):
    B, S = input_ids.shape
    V, D = word_emb.shape
    N = B * S
    info = plsc.get_sparse_core_info()
    NC, NS = info.num_cores, info.num_subcores
    NW = NC * NS

    ids3 = input_ids.reshape(NW, (N // NW) // _CH, _CH)
    gathered = _sc_gather_fn(V, D, N, NC, NS)(ids3, word_emb)

    BLK = 512
    tt3 = token_type_ids.reshape(N // BLK, 1, BLK)
    out = _tc_ln_fn(N, S, D, BLK)(
        tt3, gathered, pos_emb, type_emb, gamma.reshape(1, D), beta.reshape(1, D)
    )
    return out.reshape(B, S, D)


# speedup vs baseline: 1.7687x; 1.0173x over previous
import functools

import jax
import jax.numpy as jnp
from jax import lax
from jax.experimental import pallas as pl
from jax.experimental.pallas import tpu as pltpu
from jax.experimental.pallas import tpu_sc as plsc

_EPS = 1e-12
_CH = 128


@functools.lru_cache(maxsize=None)
def _sc_gather_fn(V, D, N, NC, NS):
    NW = NC * NS
    b_per_w = N // NW
    nch = b_per_w // _CH
    mesh = plsc.VectorSubcoreMesh(
        core_axis_name="c", subcore_axis_name="s", num_cores=NC, num_subcores=NS
    )

    @functools.partial(
        pl.kernel,
        out_type=jax.ShapeDtypeStruct((N, D), jnp.float32),
        mesh=mesh,
        scratch_types=[
            pltpu.VMEM((b_per_w,), jnp.int32),
            pltpu.VMEM((b_per_w, D), jnp.float32),
            pltpu.SemaphoreType.DMA,
        ],
    )
    def gather_kernel(ids_hbm, table_hbm, out_hbm, idx_v, rows_v, sem):
        wid = lax.axis_index("s") * NC + lax.axis_index("c")
        base = wid * b_per_w
        pltpu.sync_copy(ids_hbm.at[pl.ds(base, b_per_w)], idx_v)
        copies = [
            pltpu.async_copy(
                table_hbm.at[idx_v.at[pl.ds(j * _CH, _CH)]],
                rows_v.at[pl.ds(j * _CH, _CH)],
                sem,
            )
            for j in range(nch)
        ]
        for cp in copies:
            cp.wait()
        pltpu.sync_copy(rows_v, out_hbm.at[pl.ds(base, b_per_w)])

    return gather_kernel


@functools.lru_cache(maxsize=None)
def _tc_ln_fn(N, S, D, BLK):
    nblk = N // BLK
    spb = S // BLK
    nb = nblk // spb

    def tok_blk(j):
        return (j % nb) * spb + j // nb

    def body(tt_ref, x_ref, pos_ref, type_ref, g_ref, b_ref, o_ref):
        x = x_ref[...]
        tt = tt_ref[0, 0, :].astype(jnp.float32)[:, None]
        t0 = type_ref[0, :][None, :]
        t1 = type_ref[1, :][None, :]
        x = x + pos_ref[...] + t0 + (t1 - t0) * tt
        mean = jnp.mean(x, axis=-1, keepdims=True)
        xc = x - mean
        var = jnp.mean(xc * xc, axis=-1, keepdims=True)
        y = xc * lax.rsqrt(var + _EPS)
        o_ref[...] = y * g_ref[...] + b_ref[...]

    return pl.pallas_call(
        body,
        grid=(nblk,),
        in_specs=[
            pl.BlockSpec((1, 1, BLK), lambda i: (tok_blk(i), 0, 0)),
            pl.BlockSpec((BLK, D), lambda i: (tok_blk(i), 0)),
            pl.BlockSpec((BLK, D), lambda i: (i // nb, 0)),
            pl.BlockSpec((2, D), lambda i: (0, 0)),
            pl.BlockSpec((1, D), lambda i: (0, 0)),
            pl.BlockSpec((1, D), lambda i: (0, 0)),
        ],
        out_specs=pl.BlockSpec((BLK, D), lambda i: (tok_blk(i), 0)),
        out_shape=jax.ShapeDtypeStruct((N, D), jnp.float32),
    )


def kernel(input_ids, token_type_ids, word_emb, type_emb, pos_emb, gamma, beta):
    B, S = input_ids.shape
    V, D = word_emb.shape
    N = B * S
    info = plsc.get_sparse_core_info()
    NC, NS = info.num_cores, info.num_subcores
    NW = NC * NS

    gathered = _sc_gather_fn(V, D, N, NC, NS)(input_ids.reshape(-1), word_emb)

    BLK = 512
    tt3 = token_type_ids.reshape(N // BLK, 1, BLK)
    out = _tc_ln_fn(N, S, D, BLK)(
        tt3, gathered, pos_emb, type_emb, gamma.reshape(1, D), beta.reshape(1, D)
    )
    return out.reshape(B, S, D)


# speedup vs baseline: 2.0031x; 1.1326x over previous
import functools

import jax
import jax.numpy as jnp
from jax import lax
from jax.experimental import pallas as pl
from jax.experimental.pallas import tpu as pltpu
from jax.experimental.pallas import tpu_sc as plsc

_EPS = 1e-12
_CH = 128


@functools.lru_cache(maxsize=None)
def _sc_gather_fn(V, D, N, NC, NS):
    NW = NC * NS
    b_per_w = N // NW
    nch = b_per_w // _CH
    mesh = plsc.VectorSubcoreMesh(
        core_axis_name="c", subcore_axis_name="s", num_cores=NC, num_subcores=NS
    )

    @functools.partial(
        pl.kernel,
        out_type=jax.ShapeDtypeStruct((N, D), jnp.float32),
        mesh=mesh,
        scratch_types=[
            pltpu.VMEM((b_per_w,), jnp.int32),
            pltpu.VMEM((b_per_w, D), jnp.float32),
            pltpu.SemaphoreType.DMA,
        ],
    )
    def gather_kernel(ids_hbm, table_hbm, out_hbm, idx_v, rows_v, sem):
        wid = lax.axis_index("s") * NC + lax.axis_index("c")
        base = wid * b_per_w
        pltpu.sync_copy(ids_hbm.at[pl.ds(base, b_per_w)], idx_v)
        copies = [
            pltpu.async_copy(
                table_hbm.at[idx_v.at[pl.ds(j * _CH, _CH)]],
                rows_v.at[pl.ds(j * _CH, _CH)],
                sem,
            )
            for j in range(nch)
        ]
        for cp in copies:
            cp.wait()
        pltpu.sync_copy(rows_v, out_hbm.at[pl.ds(base, b_per_w)])

    return gather_kernel


@functools.lru_cache(maxsize=None)
def _tc_ln_fn(N, S, D, BLK):
    nblk = N // BLK
    spb = S // BLK
    nb = nblk // spb

    def tok_blk(j):
        return (j % nb) * spb + j // nb

    def body(tt_ref, x_ref, pos_ref, type_ref, g_ref, b_ref, o_ref):
        x = x_ref[...]
        tt = tt_ref[0, 0, :].astype(jnp.float32)[:, None]
        t0 = type_ref[0, :][None, :]
        t1 = type_ref[1, :][None, :]
        x = x + pos_ref[...] + t0 + (t1 - t0) * tt
        mean = jnp.mean(x, axis=-1, keepdims=True)
        xc = x - mean
        var = jnp.mean(xc * xc, axis=-1, keepdims=True)
        y = xc * lax.rsqrt(var + _EPS)
        o_ref[...] = y * g_ref[...] + b_ref[...]

    return pl.pallas_call(
        body,
        grid=(nblk,),
        in_specs=[
            pl.BlockSpec((1, 1, BLK), lambda i: (tok_blk(i), 0, 0)),
            pl.BlockSpec((BLK, D), lambda i: (tok_blk(i), 0)),
            pl.BlockSpec((BLK, D), lambda i: (i // nb, 0)),
            pl.BlockSpec((2, D), lambda i: (0, 0)),
            pl.BlockSpec((1, D), lambda i: (0, 0)),
            pl.BlockSpec((1, D), lambda i: (0, 0)),
        ],
        out_specs=pl.BlockSpec((BLK, D), lambda i: (tok_blk(i), 0)),
        out_shape=jax.ShapeDtypeStruct((N, D), jnp.float32),
    )


def kernel(input_ids, token_type_ids, word_emb, type_emb, pos_emb, gamma, beta):
    B, S = input_ids.shape
    V, D = word_emb.shape
    N = B * S
    info = plsc.get_sparse_core_info()
    NC, NS = info.num_cores, info.num_subcores
    NW = NC * NS

    gathered = _sc_gather_fn(V, D, N, NC, NS)(input_ids.reshape(-1), word_emb)

    BLK = 1024
    tt3 = token_type_ids.reshape(N // BLK, 1, BLK)
    out = _tc_ln_fn(N, S, D, BLK)(
        tt3, gathered, pos_emb, type_emb, gamma.reshape(1, D), beta.reshape(1, D)
    )
    return out.reshape(B, S, D)


# speedup vs baseline: 2.1253x; 1.0610x over previous
import functools

import jax
import jax.numpy as jnp
from jax import lax
from jax.experimental import pallas as pl
from jax.experimental.pallas import tpu as pltpu
from jax.experimental.pallas import tpu_sc as plsc

_EPS = 1e-12
_CH = 128


@functools.lru_cache(maxsize=None)
def _sc_gather_fn(V, D, N, NC, NS):
    NW = NC * NS
    b_per_w = N // NW
    nch = b_per_w // _CH
    mesh = plsc.VectorSubcoreMesh(
        core_axis_name="c", subcore_axis_name="s", num_cores=NC, num_subcores=NS
    )

    @functools.partial(
        pl.kernel,
        out_type=jax.ShapeDtypeStruct((N, D), jnp.float32),
        mesh=mesh,
        scratch_types=[
            pltpu.VMEM((b_per_w,), jnp.int32),
            pltpu.VMEM((b_per_w, D), jnp.float32),
            pltpu.SemaphoreType.DMA,
        ],
    )
    def gather_kernel(ids_hbm, table_hbm, out_hbm, idx_v, rows_v, sem):
        wid = lax.axis_index("s") * NC + lax.axis_index("c")
        base = wid * b_per_w
        pltpu.sync_copy(ids_hbm.at[pl.ds(base, b_per_w)], idx_v)
        copies = [
            pltpu.async_copy(
                table_hbm.at[idx_v.at[pl.ds(j * _CH, _CH)]],
                rows_v.at[pl.ds(j * _CH, _CH)],
                sem,
            )
            for j in range(nch)
        ]
        for cp in copies:
            cp.wait()
        pltpu.sync_copy(rows_v, out_hbm.at[pl.ds(base, b_per_w)])

    return gather_kernel


@functools.lru_cache(maxsize=None)
def _tc_ln_fn(N, S, D, BLK):
    nblk = N // BLK
    spb = S // BLK
    nb = nblk // spb

    def tok_blk(j):
        return (j % nb) * spb + j // nb

    def body(tt_ref, x_ref, pos_ref, type_ref, g_ref, b_ref, o_ref):
        x = x_ref[...]
        tt = tt_ref[0, 0, :].astype(jnp.float32)[:, None]
        t0 = type_ref[0, :][None, :]
        t1 = type_ref[1, :][None, :]
        x = x + pos_ref[...] + t0 + (t1 - t0) * tt
        mean = jnp.mean(x, axis=-1, keepdims=True)
        xc = x - mean
        var = jnp.mean(xc * xc, axis=-1, keepdims=True)
        y = xc * lax.rsqrt(var + _EPS)
        o_ref[...] = y * g_ref[...] + b_ref[...]

    return pl.pallas_call(
        body,
        grid=(nblk,),
        in_specs=[
            pl.BlockSpec((1, 1, BLK), lambda i: (tok_blk(i), 0, 0)),
            pl.BlockSpec((BLK, D), lambda i: (tok_blk(i), 0)),
            pl.BlockSpec((BLK, D), lambda i: (i // nb, 0)),
            pl.BlockSpec((2, D), lambda i: (0, 0)),
            pl.BlockSpec((1, D), lambda i: (0, 0)),
            pl.BlockSpec((1, D), lambda i: (0, 0)),
        ],
        out_specs=pl.BlockSpec((BLK, D), lambda i: (tok_blk(i), 0)),
        out_shape=jax.ShapeDtypeStruct((N, D), jnp.float32),
    )


def kernel(input_ids, token_type_ids, word_emb, type_emb, pos_emb, gamma, beta):
    B, S = input_ids.shape
    V, D = word_emb.shape
    N = B * S
    info = plsc.get_sparse_core_info()
    NC, NS = info.num_cores, info.num_subcores
    NW = NC * NS

    gathered = _sc_gather_fn(V, D, N, NC, NS)(input_ids.reshape(-1), word_emb)

    BLK = 2048
    tt3 = token_type_ids.reshape(N // BLK, 1, BLK)
    out = _tc_ln_fn(N, S, D, BLK)(
        tt3, gathered, pos_emb, type_emb, gamma.reshape(1, D), beta.reshape(1, D)
    )
    return out.reshape(B, S, D)


# speedup vs baseline: 2.1289x; 1.0017x over previous
import functools

import jax
import jax.numpy as jnp
from jax import lax
from jax.experimental import pallas as pl
from jax.experimental.pallas import tpu as pltpu
from jax.experimental.pallas import tpu_sc as plsc

_EPS = 1e-12
_CH = 128


@functools.lru_cache(maxsize=None)
def _sc_gather_fn(V, D, B, S, NC, NS):
    N = B * S
    NW = NC * NS
    b_per_w = N // NW
    nch = b_per_w // _CH
    w_per_row = S // b_per_w
    mesh = plsc.VectorSubcoreMesh(
        core_axis_name="c", subcore_axis_name="s", num_cores=NC, num_subcores=NS
    )

    @functools.partial(
        pl.kernel,
        out_type=jax.ShapeDtypeStruct((N, D), jnp.float32),
        mesh=mesh,
        scratch_types=[
            pltpu.VMEM((b_per_w,), jnp.int32),
            pltpu.VMEM((b_per_w, D), jnp.float32),
            pltpu.SemaphoreType.DMA,
        ],
    )
    def gather_kernel(ids_hbm, table_hbm, out_hbm, idx_v, rows_v, sem):
        wid = lax.axis_index("s") * NC + lax.axis_index("c")
        base = wid * b_per_w
        row = wid // w_per_row
        off = (wid % w_per_row) * b_per_w
        pltpu.sync_copy(ids_hbm.at[row, pl.ds(off, b_per_w)], idx_v)
        copies = [
            pltpu.async_copy(
                table_hbm.at[idx_v.at[pl.ds(j * _CH, _CH)]],
                rows_v.at[pl.ds(j * _CH, _CH)],
                sem,
            )
            for j in range(nch)
        ]
        for cp in copies:
            cp.wait()
        pltpu.sync_copy(rows_v, out_hbm.at[pl.ds(base, b_per_w)])

    return gather_kernel


@functools.lru_cache(maxsize=None)
def _tc_ln_fn(N, S, D, BLK):
    nblk = N // BLK
    spb = S // BLK
    nb = nblk // spb

    def tok_blk(j):
        return (j % nb) * spb + j // nb

    def body(tt_ref, x_ref, pos_ref, type_ref, g_ref, b_ref, o_ref):
        x = x_ref[...]
        tt = tt_ref[0, 0, :].astype(jnp.float32)[:, None]
        t0 = type_ref[0, :][None, :]
        t1 = type_ref[1, :][None, :]
        x = x + pos_ref[...] + t0 + (t1 - t0) * tt
        mean = jnp.mean(x, axis=-1, keepdims=True)
        xc = x - mean
        var = jnp.mean(xc * xc, axis=-1, keepdims=True)
        y = xc * lax.rsqrt(var + _EPS)
        o_ref[...] = y * g_ref[...] + b_ref[...]

    return pl.pallas_call(
        body,
        grid=(nblk,),
        in_specs=[
            pl.BlockSpec((1, 1, BLK), lambda i: (tok_blk(i), 0, 0)),
            pl.BlockSpec((BLK, D), lambda i: (tok_blk(i), 0)),
            pl.BlockSpec((BLK, D), lambda i: (i // nb, 0)),
            pl.BlockSpec((2, D), lambda i: (0, 0)),
            pl.BlockSpec((1, D), lambda i: (0, 0)),
            pl.BlockSpec((1, D), lambda i: (0, 0)),
        ],
        out_specs=pl.BlockSpec((BLK, D), lambda i: (tok_blk(i), 0)),
        out_shape=jax.ShapeDtypeStruct((N, D), jnp.float32),
    )


def kernel(input_ids, token_type_ids, word_emb, type_emb, pos_emb, gamma, beta):
    B, S = input_ids.shape
    V, D = word_emb.shape
    N = B * S
    info = plsc.get_sparse_core_info()
    NC, NS = info.num_cores, info.num_subcores
    NW = NC * NS

    gathered = _sc_gather_fn(V, D, B, S, NC, NS)(input_ids, word_emb)

    BLK = 2048
    tt3 = token_type_ids.reshape(N // BLK, 1, BLK)
    out = _tc_ln_fn(N, S, D, BLK)(
        tt3, gathered, pos_emb, type_emb, gamma.reshape(1, D), beta.reshape(1, D)
    )
    return out.reshape(B, S, D)
